# baseline (speedup 1.0000x reference)
import functools

import jax
import jax.numpy as jnp
import numpy as np
from jax.experimental import pallas as pl

_TYPE_NOUT = [("bracket", 4), ("tube", 3), ("channel", 4), ("block", 3), ("cylinder", 2), ("blockhole", 6)]
_LATENT = 32
_HIDDEN = 64
_MAXP = 6
_NT = len(_TYPE_NOUT)
_BSZ = 2048


def _body(z_ref, t_ref, w1_ref, b1_ref, w2_ref, b2_ref, mtab_ref, op_ref, om_ref):
    z = z_ref[...]
    h = jnp.dot(z, w1_ref[...], preferred_element_type=jnp.float32) + b1_ref[...]
    h = jnp.maximum(h, 0.0)
    t = t_ref[...]
    cid = jax.lax.broadcasted_iota(jnp.int32, h.shape, 1) // _HIDDEN
    hm = jnp.where(cid == t, h, 0.0)
    oh_i = jax.lax.broadcasted_iota(jnp.int32, (h.shape[0], _NT), 1)
    oh = (oh_i == t).astype(jnp.float32)
    pred = jnp.dot(hm, w2_ref[...], preferred_element_type=jnp.float32)
    op_ref[...] = pred + jnp.dot(oh, b2_ref[...], preferred_element_type=jnp.float32)
    om_ref[...] = jnp.dot(oh, mtab_ref[...], preferred_element_type=jnp.float32)


@functools.partial(jax.jit, static_argnames=())
def _run(z, t2d, w1c, b1c, w2c, b2t, mtab):
    B = z.shape[0]
    grid = (B // _BSZ,)
    const = lambda i: (0, 0)
    return pl.pallas_call(
        _body,
        grid=grid,
        in_specs=[
            pl.BlockSpec((_BSZ, _LATENT), lambda i: (i, 0)),
            pl.BlockSpec((_BSZ, 1), lambda i: (i, 0)),
            pl.BlockSpec((_LATENT, _NT * _HIDDEN), const),
            pl.BlockSpec((1, _NT * _HIDDEN), const),
            pl.BlockSpec((_NT * _HIDDEN, _MAXP), const),
            pl.BlockSpec((_NT, _MAXP), const),
            pl.BlockSpec((_NT, _MAXP), const),
        ],
        out_specs=[
            pl.BlockSpec((_BSZ, _MAXP), lambda i: (i, 0)),
            pl.BlockSpec((_BSZ, _MAXP), lambda i: (i, 0)),
        ],
        out_shape=[
            jax.ShapeDtypeStruct((B, _MAXP), jnp.float32),
            jax.ShapeDtypeStruct((B, _MAXP), jnp.float32),
        ],
    )(z, t2d, w1c, b1c, w2c, b2t, mtab)


def kernel(z, geometry_types, params):
    w1s, b1s, w2s, b2s, mrows = [], [], [], [], []
    for name, nout in _TYPE_NOUT:
        W1, b1, W2, b2 = params[name]
        w1s.append(W1)
        b1s.append(b1)
        w2s.append(jnp.pad(W2, ((0, 0), (0, _MAXP - nout))))
        b2s.append(jnp.pad(b2, (0, _MAXP - nout)))
        mrows.append(np.pad(np.ones((nout,), np.float32), (0, _MAXP - nout)))
    w1c = jnp.concatenate(w1s, axis=1)
    b1c = jnp.concatenate(b1s).reshape(1, -1)
    w2c = jnp.concatenate(w2s, axis=0)
    b2t = jnp.stack(b2s)
    mtab = jnp.asarray(np.stack(mrows))
    t2d = geometry_types.astype(jnp.int32).reshape(-1, 1)
    out_p, out_m = _run(z, t2d, w1c, b1c, w2c, b2t, mtab)
    return out_p, out_m


# speedup vs baseline: 1.6193x; 1.6193x over previous
import functools

import jax
import jax.numpy as jnp
import numpy as np
from jax.experimental import pallas as pl

_TYPE_NOUT = [("bracket", 4), ("tube", 3), ("channel", 4), ("block", 3), ("cylinder", 2), ("blockhole", 6)]
_LATENT = 32
_HIDDEN = 64
_MAXP = 6
_NT = len(_TYPE_NOUT)
_OUTP = 8
_BSZ = 2048


def _body(z_ref, t_ref, w1t_ref, b1t_ref, w2t_ref, b2t_ref, mt_ref, op_ref, om_ref):
    z = z_ref[...]
    w1t = w1t_ref[...]
    hT = jax.lax.dot_general(
        w1t, z, (((1,), (1,)), ((), ())),
        preferred_element_type=jnp.float32)
    hT = jnp.maximum(hT + b1t_ref[...], 0.0)
    t = t_ref[...].reshape(1, -1)
    accp = jnp.zeros(op_ref.shape, jnp.float32)
    accm = jnp.zeros(om_ref.shape, jnp.float32)
    for ty in range(_NT):
        pt = jnp.dot(w2t_ref[ty * _OUTP:(ty + 1) * _OUTP, :],
                     hT[ty * _HIDDEN:(ty + 1) * _HIDDEN, :],
                     preferred_element_type=jnp.float32)
        sel = (t == ty).astype(jnp.float32)
        accp = accp + sel * (pt + b2t_ref[:, ty:ty + 1])
        accm = accm + sel * mt_ref[:, ty:ty + 1]
    op_ref[...] = accp
    om_ref[...] = accm


@jax.jit
def _run(z, t3d, w1t, b1t, w2t, b2t, mt):
    B = z.shape[0]
    nblk = B // _BSZ
    const = lambda i: (0, 0)
    opT, omT = pl.pallas_call(
        _body,
        grid=(nblk,),
        in_specs=[
            pl.BlockSpec((_BSZ, _LATENT), lambda i: (i, 0)),
            pl.BlockSpec((1, 1, _BSZ), lambda i: (i, 0, 0)),
            pl.BlockSpec((_NT * _HIDDEN, _LATENT), const),
            pl.BlockSpec((_NT * _HIDDEN, 1), const),
            pl.BlockSpec((_NT * _OUTP, _HIDDEN), const),
            pl.BlockSpec((_OUTP, _NT), const),
            pl.BlockSpec((_OUTP, _NT), const),
        ],
        out_specs=[
            pl.BlockSpec((_OUTP, _BSZ), lambda i: (0, i)),
            pl.BlockSpec((_OUTP, _BSZ), lambda i: (0, i)),
        ],
        out_shape=[
            jax.ShapeDtypeStruct((_OUTP, B), jnp.float32),
            jax.ShapeDtypeStruct((_OUTP, B), jnp.float32),
        ],
    )(z, t3d, w1t, b1t, w2t, b2t, mt)
    return opT[:_MAXP].T, omT[:_MAXP].T


def kernel(z, geometry_types, params):
    w1s, b1s, w2s, b2s, mrows = [], [], [], [], []
    for name, nout in _TYPE_NOUT:
        W1, b1, W2, b2 = params[name]
        w1s.append(W1.T)
        b1s.append(b1)
        w2s.append(jnp.pad(W2.T, ((0, _OUTP - nout), (0, 0))))
        b2s.append(jnp.pad(b2, (0, _OUTP - nout)))
        mrows.append(np.pad(np.ones((nout,), np.float32), (0, _OUTP - nout)))
    w1t = jnp.concatenate(w1s, axis=0)
    b1t = jnp.concatenate(b1s).reshape(-1, 1)
    w2t = jnp.concatenate(w2s, axis=0)
    b2t = jnp.stack(b2s, axis=1)
    mt = jnp.asarray(np.stack(mrows, axis=1))
    B = z.shape[0]
    t3d = geometry_types.astype(jnp.int32).reshape(B // _BSZ, 1, _BSZ)
    out_p, out_m = _run(z, t3d, w1t, b1t, w2t, b2t, mt)
    return out_p, out_m
